# baseline (device time: 14256 ns/iter reference)
import jax
import jax.numpy as jnp
from jax import lax
from jax.experimental import pallas as pl
from jax.experimental.pallas import tpu as pltpu

N_DEV = 8
BLK = 128

SEND_ORDER = (2, 6, 3, 5, 1, 7, 4)
RECV_ORDER = (4, 1, 7, 3, 5, 2, 6)


def kernel(x, w_mat):
    m, k_per = x.shape
    k, n = w_mat.shape
    assert m == N_DEV * BLK and k_per == BLK and k == N_DEV * BLK

    def body(x_ref, w_hbm, out_ref, comm_ref, w_ref, send_sems, recv_sems,
             w_sem, ready_sems):
        my = lax.axis_index("i")

        barrier_sem = pltpu.get_barrier_semaphore()
        pl.semaphore_signal(barrier_sem, 1)

        for off in range(1, N_DEV):
            tgt = lax.rem(my + off, N_DEV)
            pl.semaphore_signal(
                ready_sems.at[N_DEV - 1 - off], inc=1,
                device_id=(tgt,), device_id_type=pl.DeviceIdType.MESH,
            )

        w_copy = pltpu.make_async_copy(w_hbm, w_ref, w_sem)
        w_copy.start()

        pl.semaphore_wait(barrier_sem, 1)

        rdmas = {}
        for off in SEND_ORDER:
            dst = lax.rem(my + off, N_DEV)
            pl.semaphore_wait(ready_sems.at[off - 1], 1)
            rdma = pltpu.make_async_remote_copy(
                src_ref=x_ref.at[pl.ds(dst * BLK, BLK), :],
                dst_ref=comm_ref.at[off - 1],
                send_sem=send_sems.at[off - 1],
                recv_sem=recv_sems.at[off - 1],
                device_id=(dst,),
                device_id_type=pl.DeviceIdType.MESH,
            )
            rdma.start()
            rdmas[off] = rdma

        w_copy.wait()
        acc = jnp.dot(
            x_ref[pl.ds(my * BLK, BLK), :],
            w_ref[pl.ds(my * BLK, BLK), :],
            preferred_element_type=jnp.float32,
        )
        for off in RECV_ORDER:
            rdmas[off].wait_recv()
            src = lax.rem(my + N_DEV - off, N_DEV)
            acc += jnp.dot(
                comm_ref[off - 1],
                w_ref[pl.ds(src * BLK, BLK), :],
                preferred_element_type=jnp.float32,
            )

        c = 0.7978845608028654
        out_ref[:, :] = 0.5 * acc * (1.0 + jnp.tanh(c * (acc + 0.044715 * acc**3)))

        for off in SEND_ORDER:
            rdmas[off].wait_send()

    return pl.pallas_call(
        body,
        out_shape=jax.ShapeDtypeStruct((BLK, n), jnp.float32),
        in_specs=[
            pl.BlockSpec(memory_space=pltpu.VMEM),
            pl.BlockSpec(memory_space=pltpu.MemorySpace.HBM),
        ],
        out_specs=pl.BlockSpec(memory_space=pltpu.VMEM),
        scratch_shapes=[
            pltpu.VMEM((N_DEV - 1, BLK, BLK), x.dtype),
            pltpu.VMEM((N_DEV * BLK, n), w_mat.dtype),
            pltpu.SemaphoreType.DMA((N_DEV - 1,)),
            pltpu.SemaphoreType.DMA((N_DEV - 1,)),
            pltpu.SemaphoreType.DMA,
            pltpu.SemaphoreType.REGULAR((N_DEV - 1,)),
        ],
        compiler_params=pltpu.CompilerParams(collective_id=0),
    )(x, w_mat)


# device time: 12777 ns/iter; 1.1158x vs baseline; 1.1158x over previous
import jax
import jax.numpy as jnp
from jax import lax
from jax.experimental import pallas as pl
from jax.experimental.pallas import tpu as pltpu

N_DEV = 8
BLK = 128

SEND_ORDER = (2, 6, 3, 5, 1, 7, 4)
RECV_ORDER = (4, 1, 7, 3, 5, 2, 6)


def kernel(x, w_mat):
    m, k_per = x.shape
    k, n = w_mat.shape
    assert m == N_DEV * BLK and k_per == BLK and k == N_DEV * BLK

    def body(x_ref, w_hbm, out_ref, sendbuf_ref, comm_ref, w_ref,
             send_sems, recv_sems, w_sem):
        my = lax.axis_index("i")

        barrier_sem = pltpu.get_barrier_semaphore()
        for off in range(1, N_DEV):
            tgt = lax.rem(my + off, N_DEV)
            pl.semaphore_signal(
                barrier_sem, inc=1,
                device_id=(tgt,), device_id_type=pl.DeviceIdType.MESH,
            )

        w_copy = pltpu.make_async_copy(w_hbm, w_ref, w_sem)
        w_copy.start()

        for off in SEND_ORDER:
            dst = lax.rem(my + off, N_DEV)
            sendbuf_ref[off - 1, :, :] = x_ref[
                pl.ds(dst * BLK, BLK), :
            ].astype(jnp.bfloat16)

        pl.semaphore_wait(barrier_sem, N_DEV - 1)

        rdmas = {}
        for off in SEND_ORDER:
            dst = lax.rem(my + off, N_DEV)
            rdma = pltpu.make_async_remote_copy(
                src_ref=sendbuf_ref.at[off - 1],
                dst_ref=comm_ref.at[off - 1],
                send_sem=send_sems.at[off - 1],
                recv_sem=recv_sems.at[off - 1],
                device_id=(dst,),
                device_id_type=pl.DeviceIdType.MESH,
            )
            rdma.start()
            rdmas[off] = rdma

        w_copy.wait()
        acc = jnp.dot(
            x_ref[pl.ds(my * BLK, BLK), :],
            w_ref[pl.ds(my * BLK, BLK), :],
            preferred_element_type=jnp.float32,
        )
        for off in RECV_ORDER:
            rdmas[off].wait_recv()
            src = lax.rem(my + N_DEV - off, N_DEV)
            acc += jnp.dot(
                comm_ref[off - 1],
                w_ref[pl.ds(src * BLK, BLK), :],
                preferred_element_type=jnp.float32,
            )

        c = 0.7978845608028654
        out_ref[:, :] = 0.5 * acc * (1.0 + jnp.tanh(c * (acc + 0.044715 * acc**3)))

        for off in SEND_ORDER:
            rdmas[off].wait_send()

    return pl.pallas_call(
        body,
        out_shape=jax.ShapeDtypeStruct((BLK, n), jnp.float32),
        in_specs=[
            pl.BlockSpec(memory_space=pltpu.VMEM),
            pl.BlockSpec(memory_space=pltpu.MemorySpace.HBM),
        ],
        out_specs=pl.BlockSpec(memory_space=pltpu.VMEM),
        scratch_shapes=[
            pltpu.VMEM((N_DEV - 1, BLK, BLK), jnp.bfloat16),
            pltpu.VMEM((N_DEV - 1, BLK, BLK), jnp.bfloat16),
            pltpu.VMEM((N_DEV * BLK, n), w_mat.dtype),
            pltpu.SemaphoreType.DMA((N_DEV - 1,)),
            pltpu.SemaphoreType.DMA((N_DEV - 1,)),
            pltpu.SemaphoreType.DMA,
        ],
        compiler_params=pltpu.CompilerParams(collective_id=0),
    )(x, w_mat)
